# baseline (speedup 1.0000x reference)
import jax
import jax.numpy as jnp
from jax.experimental import pallas as pl

_ITERATIONS = 2
_NEG_INF = float("-inf")
_HALO = 15


def _shift(x, d, axis):
    n = x.shape[axis]
    pad_shape = list(x.shape)
    pad_shape[axis] = abs(d)
    pad = jnp.full(tuple(pad_shape), _NEG_INF, x.dtype)
    lo = [slice(None)] * x.ndim
    hi = [slice(None)] * x.ndim
    lo[axis] = slice(0, n - abs(d))
    hi[axis] = slice(abs(d), None)
    if d > 0:
        return jnp.concatenate([pad, x[tuple(lo)]], axis=axis)
    return jnp.concatenate([x[tuple(hi)], pad], axis=axis)


def _maxpool1d(x, axis):
    s = jnp.maximum(x, _shift(x, 1, axis))
    s = jnp.maximum(s, _shift(s, 2, axis))
    t = jnp.maximum(x, _shift(x, -1, axis))
    t = jnp.maximum(t, _shift(t, -2, axis))
    return jnp.maximum(s, t)


def _maxpool(x):
    return _maxpool1d(_maxpool1d(x, 2), 0)


def _nms_strip(x):
    max_mask = x == _maxpool(x)
    for _ in range(_ITERATIONS):
        supp_mask = _maxpool(max_mask.astype(jnp.bfloat16)) > 0
        supp_scores = jnp.where(supp_mask, 0.0, x)
        new_max = (supp_scores == _maxpool(supp_scores)) & (~supp_mask)
        max_mask = max_mask | new_max
    return jnp.where(max_mask, x, 0.0)


def _nms_kernel(s_ref, o_ref):
    n_img, _, h, w = s_ref.shape
    strip = 128
    for s0 in range(0, h, strip):
        lo = max(0, s0 - _HALO)
        hi = min(h, s0 + strip + _HALO)
        xs = s_ref[:, 0, lo:hi, :]
        xt = jnp.transpose(xs, (1, 0, 2))
        res = _nms_strip(xt)[s0 - lo : s0 - lo + strip]
        o_ref[:, 0, s0 : s0 + strip, :] = jnp.transpose(res, (1, 0, 2))


def kernel(scores):
    b, c, h, w = scores.shape
    per_step = 8
    return pl.pallas_call(
        _nms_kernel,
        grid=(b // per_step,),
        in_specs=[pl.BlockSpec((per_step, 1, h, w), lambda i: (i, 0, 0, 0))],
        out_specs=pl.BlockSpec((per_step, 1, h, w), lambda i: (i, 0, 0, 0)),
        out_shape=jax.ShapeDtypeStruct(scores.shape, scores.dtype),
    )(scores)


# speedup vs baseline: 1.6603x; 1.6603x over previous
import jax
import jax.numpy as jnp
from jax.experimental import pallas as pl

_ITERATIONS = 2
_NEG_INF = float("-inf")


def _shift(x, d, axis):
    n = x.shape[axis]
    if axis == 0:
        pad = jnp.full((abs(d), x.shape[1]), _NEG_INF, x.dtype)
        if d > 0:
            return jnp.concatenate([pad, x[: n - d, :]], axis=0)
        return jnp.concatenate([x[-d:, :], pad], axis=0)
    pad = jnp.full((x.shape[0], abs(d)), _NEG_INF, x.dtype)
    if d > 0:
        return jnp.concatenate([pad, x[:, : n - d]], axis=1)
    return jnp.concatenate([x[:, -d:], pad], axis=1)


def _maxpool(x):
    s = jnp.maximum(x, _shift(x, 1, 1))
    s = jnp.maximum(s, _shift(s, 2, 1))
    t = jnp.maximum(x, _shift(x, -1, 1))
    t = jnp.maximum(t, _shift(t, -2, 1))
    y = jnp.maximum(s, t)
    a = jnp.maximum(y, _shift(y, 1, 0))
    a = jnp.maximum(a, _shift(a, 2, 0))
    return jnp.maximum(a, _shift(a, -3, 0))


def _nms_kernel(s_ref, o_ref):
    h = s_ref.shape[2]
    x = s_ref[0, 0]
    pad_f32 = jnp.full((3, x.shape[1]), _NEG_INF, x.dtype)
    xp = jnp.concatenate([x, pad_f32], axis=0)
    max_mask = xp == _maxpool(xp)
    for _ in range(_ITERATIONS):
        supp_mask = _maxpool(max_mask.astype(jnp.bfloat16)) > 0
        supp_scores = jnp.concatenate(
            [jnp.where(supp_mask[:h], 0.0, x), pad_f32], axis=0
        )
        new_max = (supp_scores == _maxpool(supp_scores)) & (~supp_mask)
        max_mask = max_mask | new_max
    o_ref[0, 0] = jnp.where(max_mask[:h], x, 0.0)


def kernel(scores):
    b, c, h, w = scores.shape
    return pl.pallas_call(
        _nms_kernel,
        grid=(b * c,),
        in_specs=[pl.BlockSpec((1, 1, h, w), lambda i: (i, 0, 0, 0))],
        out_specs=pl.BlockSpec((1, 1, h, w), lambda i: (i, 0, 0, 0)),
        out_shape=jax.ShapeDtypeStruct(scores.shape, scores.dtype),
    )(scores)


# speedup vs baseline: 1.7374x; 1.0464x over previous
import jax
import jax.numpy as jnp
from jax.experimental import pallas as pl

_ITERATIONS = 2
_NEG_INF = float("-inf")


def _shift(x, d, axis):
    n = x.shape[axis]
    if axis == 0:
        pad = jnp.full((abs(d), x.shape[1]), _NEG_INF, x.dtype)
        if d > 0:
            return jnp.concatenate([pad, x[: n - d, :]], axis=0)
        return jnp.concatenate([x[-d:, :], pad], axis=0)
    pad = jnp.full((x.shape[0], abs(d)), _NEG_INF, x.dtype)
    if d > 0:
        return jnp.concatenate([pad, x[:, : n - d]], axis=1)
    return jnp.concatenate([x[:, -d:], pad], axis=1)


def _maxpool(x):
    a = jnp.maximum(x, _shift(x, 1, 0))
    a = jnp.maximum(a, _shift(a, 2, 0))
    y = jnp.maximum(a, _shift(a, -3, 0))
    s = jnp.maximum(y, _shift(y, 1, 1))
    s = jnp.maximum(s, _shift(s, 2, 1))
    t = jnp.maximum(y, _shift(y, -1, 1))
    t = jnp.maximum(t, _shift(t, -2, 1))
    return jnp.maximum(s, t)


def _nms_kernel(s_ref, o_ref):
    h = s_ref.shape[2]
    x = s_ref[0, 0]
    pad_f32 = jnp.full((3, x.shape[1]), _NEG_INF, x.dtype)
    xp = jnp.concatenate([x, pad_f32], axis=0)
    max_mask = xp == _maxpool(xp)
    for _ in range(_ITERATIONS):
        supp_mask = _maxpool(max_mask.astype(jnp.bfloat16)) > 0
        supp_scores = jnp.concatenate(
            [jnp.where(supp_mask[:h], 0.0, x), pad_f32], axis=0
        )
        new_max = (supp_scores == _maxpool(supp_scores)) & (~supp_mask)
        max_mask = max_mask | new_max
    o_ref[0, 0] = jnp.where(max_mask[:h], x, 0.0)


def kernel(scores):
    b, c, h, w = scores.shape
    return pl.pallas_call(
        _nms_kernel,
        grid=(b * c,),
        in_specs=[pl.BlockSpec((1, 1, h, w), lambda i: (i, 0, 0, 0))],
        out_specs=pl.BlockSpec((1, 1, h, w), lambda i: (i, 0, 0, 0)),
        out_shape=jax.ShapeDtypeStruct(scores.shape, scores.dtype),
    )(scores)
